# baseline (speedup 1.0000x reference)
import jax
import jax.numpy as jnp
from jax import lax
from jax.experimental import pallas as pl
from jax.experimental.pallas import tpu as pltpu
from jax.experimental.pallas import tpu_sc as plsc

N, K, D = 32768, 1024, 64
ROWS_PER_BLOCK = 1024
NUM_BLOCKS = N // ROWS_PER_BLOCK
BLOCKS_LO = 25
BLOCKS_HI = NUM_BLOCKS - BLOCKS_LO
NW = 32
CG = N // 128
CG_LO = BLOCKS_LO * (ROWS_PER_BLOCK // 128)
CG_HI = CG - CG_LO
DG = D // 8


def _argmax_body(x_ref, idx_ref):
    xb = x_ref[...]
    m = jnp.max(xb, axis=-1, keepdims=True)
    col = lax.broadcasted_iota(jnp.int32, xb.shape, 1)
    cand = jnp.where(xb == m, col, K)
    am = jnp.min(cand, axis=-1)
    idx_ref[...] = am.reshape(ROWS_PER_BLOCK // 128, 128)


def _tc_argmax_part(x, boff, nblocks):
    return pl.pallas_call(
        _argmax_body,
        grid=(nblocks,),
        in_specs=[
            pl.BlockSpec(
                (ROWS_PER_BLOCK, K),
                lambda b, boff=boff: (boff + b, 0),
            )
        ],
        out_specs=pl.BlockSpec((ROWS_PER_BLOCK // 128, 128), lambda b: (b, 0)),
        out_shape=jax.ShapeDtypeStruct(
            (nblocks * (ROWS_PER_BLOCK // 128), 128), jnp.int32
        ),
    )(x)


def _gather_loop(w_hbm, idx_hbm, out_hbm, w_v, idx_v, outv0, outv1, sem, ncg, cbase):
    wid = lax.axis_index("s") * 2 + lax.axis_index("c")
    d0 = wid * 2
    rg = wid // 4
    s0 = (wid % 4) * 2
    pltpu.sync_copy(w_hbm.at[pl.ds(d0, 2)], w_v)
    pltpu.sync_copy(idx_hbm, idx_v)
    row0 = jnp.zeros((16,), jnp.int32)
    row1 = row0 + 1
    half = ncg // 2
    copies = []
    for hpart in range(2):

        @plsc.parallel_loop(hpart * half, (hpart + 1) * half, unroll=8)
        def cg_body(cg):
            for g in range(8):
                tok = idx_v[cg, pl.ds(g * 16, 16)]
                v0 = plsc.load_gather(w_v, [row0, tok])
                v1 = plsc.load_gather(w_v, [row1, tok])
                outv0[cg, 0, pl.ds(g * 16, 16)] = v0
                outv1[cg, 0, pl.ds(g * 16, 16)] = v1

        src0 = outv0.at[pl.ds(hpart * half, half)]
        src1 = outv1.at[pl.ds(hpart * half, half)]
        cdst = pl.ds(cbase + hpart * half, half)
        copies.append(pltpu.async_copy(src0, out_hbm.at[rg, cdst, pl.ds(s0, 1), :], sem))
        copies.append(pltpu.async_copy(src1, out_hbm.at[rg, cdst, pl.ds(s0 + 1, 1), :], sem))
    for c in copies:
        c.wait()


_SC_PARAMS = dict(
    compiler_params=pltpu.CompilerParams(
        use_tc_tiling_on_sc=False, needs_layout_passes=False
    ),
)


def _sc_scratch(ncg):
    return [
        pltpu.VMEM((2, K), jnp.float32),
        pltpu.VMEM((ncg, 128), jnp.int32),
        pltpu.VMEM((ncg, 1, 128), jnp.float32),
        pltpu.VMEM((ncg, 1, 128), jnp.float32),
        pltpu.SemaphoreType.DMA,
    ]


def _sc_gather_lo(W, idx2):
    def body(w_hbm, idx_hbm, out_hbm, w_v, idx_v, outv0, outv1, sem):
        _gather_loop(w_hbm, idx_hbm, out_hbm, w_v, idx_v, outv0, outv1, sem, CG_LO, 0)

    mesh = plsc.VectorSubcoreMesh(core_axis_name="c", subcore_axis_name="s")
    run = pl.kernel(
        body,
        out_type=jax.ShapeDtypeStruct((DG, CG, 8, 128), jnp.float32),
        mesh=mesh,
        scratch_types=_sc_scratch(CG_LO),
        **_SC_PARAMS,
    )
    return run(W, idx2)


def _sc_gather_hi(W, idx2, out_ref):
    def body(w_hbm, idx_hbm, out_hbm, w_v, idx_v, outv0, outv1, sem):
        _gather_loop(
            w_hbm, idx_hbm, out_hbm, w_v, idx_v, outv0, outv1, sem, CG_HI, CG_LO
        )

    mesh = plsc.VectorSubcoreMesh(core_axis_name="c", subcore_axis_name="s")
    run = pl.kernel(
        body,
        out_type=(),
        mesh=mesh,
        scratch_types=_sc_scratch(CG_HI),
        **_SC_PARAMS,
    )
    run(W, idx2, out_ref)


def kernel(x, W):
    idx_lo = _tc_argmax_part(x, 0, BLOCKS_LO)
    out4_lo = _sc_gather_lo(W, idx_lo)
    idx_hi = _tc_argmax_part(x, BLOCKS_LO, BLOCKS_HI)
    out_ref = jax.new_ref(out4_lo)
    _sc_gather_hi(W, idx_hi, out_ref)
    out4 = out_ref[...]
    return out4.transpose(0, 2, 1, 3).reshape(D, N).T


# speedup vs baseline: 1.0055x; 1.0055x over previous
import jax
import jax.numpy as jnp
from jax import lax
from jax.experimental import pallas as pl
from jax.experimental.pallas import tpu as pltpu
from jax.experimental.pallas import tpu_sc as plsc

N, K, D = 32768, 1024, 64
ROWS_PER_BLOCK = 1024
NUM_BLOCKS = N // ROWS_PER_BLOCK
BLOCKS_LO = 25
BLOCKS_HI = NUM_BLOCKS - BLOCKS_LO
NW = 32
CG = N // 128
CG_LO = BLOCKS_LO * (ROWS_PER_BLOCK // 128)
CG_HI = CG - CG_LO
DG = D // 8


def _argmax_body(x_ref, idx_ref):
    xb = x_ref[...]
    m = jnp.max(xb, axis=-1, keepdims=True)
    col = lax.broadcasted_iota(jnp.int32, xb.shape, 1)
    cand = jnp.where(xb == m, col, K)
    am = jnp.min(cand, axis=-1)
    idx_ref[...] = am.reshape(ROWS_PER_BLOCK // 128, 128)


def _tc_argmax_part(x, boff, nblocks):
    return pl.pallas_call(
        _argmax_body,
        grid=(nblocks,),
        in_specs=[
            pl.BlockSpec(
                (ROWS_PER_BLOCK, K),
                lambda b, boff=boff: (boff + b, 0),
            )
        ],
        out_specs=pl.BlockSpec((ROWS_PER_BLOCK // 128, 128), lambda b: (b, 0)),
        out_shape=jax.ShapeDtypeStruct(
            (nblocks * (ROWS_PER_BLOCK // 128), 128), jnp.int32
        ),
    )(x)


def _gather_loop(w_hbm, idx_hbm, w_v, idx_v, outv0, outv1, ncg):
    wid = lax.axis_index("s") * 2 + lax.axis_index("c")
    d0 = wid * 2
    pltpu.sync_copy(w_hbm.at[pl.ds(d0, 2)], w_v)
    pltpu.sync_copy(idx_hbm, idx_v)
    row0 = jnp.zeros((16,), jnp.int32)
    row1 = row0 + 1

    @plsc.parallel_loop(0, ncg, unroll=8)
    def cg_body(cg):
        for g in range(8):
            tok = idx_v[cg, pl.ds(g * 16, 16)]
            v0 = plsc.load_gather(w_v, [row0, tok])
            v1 = plsc.load_gather(w_v, [row1, tok])
            outv0[cg, 0, pl.ds(g * 16, 16)] = v0
            outv1[cg, 0, pl.ds(g * 16, 16)] = v1
    rg = wid // 4
    s0 = (wid % 4) * 2
    return rg, s0


_SC_PARAMS = dict(
    compiler_params=pltpu.CompilerParams(
        use_tc_tiling_on_sc=False, needs_layout_passes=False
    ),
)


def _sc_scratch(ncg):
    return [
        pltpu.VMEM((2, K), jnp.float32),
        pltpu.VMEM((ncg, 128), jnp.int32),
        pltpu.VMEM((ncg, 1, 128), jnp.float32),
        pltpu.VMEM((ncg, 1, 128), jnp.float32),
        pltpu.SemaphoreType.DMA,
    ]


def _sc_gather_lo(W, idx2):
    def body(w_hbm, idx_hbm, out_hbm, w_v, idx_v, outv0, outv1, sem):
        rg, s0 = _gather_loop(w_hbm, idx_hbm, w_v, idx_v, outv0, outv1, CG_LO)
        pltpu.sync_copy(outv0, out_hbm.at[rg, pl.ds(0, CG_LO), pl.ds(s0, 1), :])
        pltpu.sync_copy(outv1, out_hbm.at[rg, pl.ds(0, CG_LO), pl.ds(s0 + 1, 1), :])

    mesh = plsc.VectorSubcoreMesh(core_axis_name="c", subcore_axis_name="s")
    run = pl.kernel(
        body,
        out_type=jax.ShapeDtypeStruct((DG, CG, 8, 128), jnp.float32),
        mesh=mesh,
        scratch_types=_sc_scratch(CG_LO),
        **_SC_PARAMS,
    )
    return run(W, idx2)


def _sc_gather_hi(W, idx2, out_ref):
    def body(w_hbm, idx_hbm, out_hbm, w_v, idx_v, outv0, outv1, sem):
        rg, s0 = _gather_loop(w_hbm, idx_hbm, w_v, idx_v, outv0, outv1, CG_HI)
        pltpu.sync_copy(outv0, out_hbm.at[rg, pl.ds(CG_LO, CG_HI), pl.ds(s0, 1), :])
        pltpu.sync_copy(outv1, out_hbm.at[rg, pl.ds(CG_LO, CG_HI), pl.ds(s0 + 1, 1), :])

    mesh = plsc.VectorSubcoreMesh(core_axis_name="c", subcore_axis_name="s")
    run = pl.kernel(
        body,
        out_type=(),
        mesh=mesh,
        scratch_types=_sc_scratch(CG_HI),
        **_SC_PARAMS,
    )
    run(W, idx2, out_ref)


def kernel(x, W):
    idx_lo = _tc_argmax_part(x, 0, BLOCKS_LO)
    out4_lo = _sc_gather_lo(W, idx_lo)
    idx_hi = _tc_argmax_part(x, BLOCKS_LO, BLOCKS_HI)
    out_ref = jax.new_ref(out4_lo)
    _sc_gather_hi(W, idx_hi, out_ref)
    out4 = out_ref[...]
    return out4.transpose(0, 2, 1, 3).reshape(D, N).T
